# baseline (speedup 1.0000x reference)
import functools

import jax
import jax.numpy as jnp
from jax import lax
from jax.experimental import pallas as pl
from jax.experimental.pallas import tpu as pltpu
from jax.experimental.pallas import tpu_sc as plsc

OFF = 2600000
NONE_HOT = 26
NMULTI = 200
EMB = 16
PACK = 128 // EMB
CHUNK = 128
PACK_R = 4096
BM = 512


def _pack_body(wt_ref, out_ref):
    cat = jnp.concatenate(
        [wt_ref[:, pl.ds(PACK_R * u, PACK_R)] for u in range(PACK)], axis=0)
    out_ref[...] = cat.T


@functools.cache
def _make_pack(nblk):
    return pl.pallas_call(
        _pack_body,
        grid=(nblk,),
        in_specs=[
            pl.BlockSpec((EMB, PACK * PACK_R), lambda i: (0, i)),
        ],
        out_specs=pl.BlockSpec((PACK_R, 128), lambda i: (i, 0)),
        out_shape=jax.ShapeDtypeStruct((nblk * PACK_R, 128), jnp.float32),
    )


def _sc_fc_body(r_hbm, ftab1, fc_hbm, r_v, f0, f1, s0, s1):
    nch = r_hbm.shape[0] // 32
    wid = lax.axis_index("s") * 2 + lax.axis_index("c")
    pltpu.sync_copy(r_hbm.at[pl.ds(wid * nch, nch)], r_v)
    base = wid * nch * CHUNK

    def copy(c, fb, sem):
        return pltpu.make_async_copy(ftab1.at[r_v.at[c]], fb, sem)

    copy(0, f0, s0).start()

    def body(k, carry):
        c0 = 2 * k
        copy(c0 + 1, f1, s1).start()
        copy(c0, f0, s0).wait()
        pltpu.sync_copy(f0, fc_hbm.at[pl.ds(base + c0 * CHUNK, CHUNK)])

        @pl.when(c0 + 2 < nch)
        def _():
            copy(c0 + 2, f0, s0).start()

        copy(c0 + 1, f1, s1).wait()
        pltpu.sync_copy(f1, fc_hbm.at[pl.ds(base + (c0 + 1) * CHUNK, CHUNK)])
        return carry

    lax.fori_loop(0, nch // 2, body, 0)


@functools.cache
def _make_sc_fc(n_idx):
    nch = n_idx // CHUNK // 32
    mesh = plsc.VectorSubcoreMesh(core_axis_name="c", subcore_axis_name="s")
    return pl.kernel(
        _sc_fc_body,
        mesh=mesh,
        compiler_params=pltpu.CompilerParams(use_tc_tiling_on_sc=False),
        out_type=jax.ShapeDtypeStruct((n_idx, 1), jnp.float32),
        scratch_types=[
            pltpu.VMEM((nch, CHUNK), jnp.int32),
            pltpu.VMEM((CHUNK, 1), jnp.float32),
            pltpu.VMEM((CHUNK, 1), jnp.float32),
            pltpu.SemaphoreType.DMA,
            pltpu.SemaphoreType.DMA,
        ],
    )


def _sc_gather_body(gidx_hbm, r_hbm, c_hbm, emb_hbm,
                    gid_v, r_v, be0, be1, ext, se0, se1):
    nch = gidx_hbm.shape[0] // 32
    wid = lax.axis_index("s") * 2 + lax.axis_index("c")
    pltpu.sync_copy(gidx_hbm.at[pl.ds(wid * nch, nch)], gid_v)
    pltpu.sync_copy(r_hbm.at[pl.ds(wid * nch, nch)], r_v)
    base = wid * nch * CHUNK
    i16 = lax.iota(jnp.int32, 16)

    def gather(c, be, se):
        return pltpu.make_async_copy(c_hbm.at[gid_v.at[c]], be, se)

    def process(c, be):
        for g in range(CHUNK // 16):
            r16 = r_v[c, pl.ds(g * 16, 16)]
            rows = g * 16 + i16
            col0 = ((r16 >> (PACK_R.bit_length() - 1)) & (PACK - 1)) * EMB
            for cc in range(EMB):
                v = plsc.load_gather(be, [rows, col0 + cc])
                plsc.store_scatter(ext, [i16 * EMB + (g * 16 * EMB + cc)], v)
        pltpu.sync_copy(
            ext, emb_hbm.at[pl.ds((base + c * CHUNK) * EMB, CHUNK * EMB)])

    gather(0, be0, se0).start()

    def body(k, carry):
        c0 = 2 * k
        gather(c0 + 1, be1, se1).start()
        gather(c0, be0, se0).wait()
        process(c0, be0)

        @pl.when(c0 + 2 < nch)
        def _():
            gather(c0 + 2, be0, se0).start()

        gather(c0 + 1, be1, se1).wait()
        process(c0 + 1, be1)
        return carry

    lax.fori_loop(0, nch // 2, body, 0)


@functools.cache
def _make_sc_gather(n_idx):
    nch = n_idx // CHUNK // 32
    mesh = plsc.VectorSubcoreMesh(core_axis_name="c", subcore_axis_name="s")
    return pl.kernel(
        _sc_gather_body,
        mesh=mesh,
        compiler_params=pltpu.CompilerParams(needs_layout_passes=False),
        out_type=jax.ShapeDtypeStruct((n_idx * EMB,), jnp.float32),
        scratch_types=[
            pltpu.VMEM((nch, CHUNK), jnp.int32),
            pltpu.VMEM((nch, CHUNK), jnp.int32),
            pltpu.VMEM((CHUNK, 128), jnp.float32),
            pltpu.VMEM((CHUNK, 128), jnp.float32),
            pltpu.VMEM((CHUNK * EMB,), jnp.float32),
            pltpu.SemaphoreType.DMA,
            pltpu.SemaphoreType.DMA,
        ],
    )


def _tc_body(xm_ref, emb_ref, fc_ref, wm_ref, wf_ref, pe_ref, pf_ref, a_ref,
             w1a_ref, w1b_ref, b1_ref, w2_ref, b2_ref, w3_ref, b3_ref,
             w4_ref, b4_ref, out_ref):
    f32 = jnp.float32
    m = (xm_ref[...] != 0).astype(f32)
    me = jnp.dot(m, wm_ref[...], preferred_element_type=f32)
    mf = jnp.dot(m, wf_ref[...], preferred_element_type=f32)
    npad = float(NMULTI) - jnp.sum(m, axis=1, keepdims=True)
    me = me + npad * pe_ref[...]
    mf = mf + npad * pf_ref[...]

    emb = emb_ref[...]
    a = a_ref[...]
    s = jnp.dot(emb, a, preferred_element_type=f32) + me
    sq = jnp.dot(emb * emb, a, preferred_element_type=f32) + me * me
    fm = (jnp.sum(fc_ref[...], axis=1, keepdims=True) + mf
          + 0.5 * jnp.sum(s * s - sq, axis=1, keepdims=True))

    h = jnp.maximum(jnp.dot(emb, w1a_ref[...], preferred_element_type=f32)
                    + jnp.dot(me, w1b_ref[...], preferred_element_type=f32)
                    + b1_ref[...], 0.0)
    h = jnp.maximum(jnp.dot(h, w2_ref[...], preferred_element_type=f32)
                    + b2_ref[...], 0.0)
    h = jnp.maximum(jnp.dot(h, w3_ref[...], preferred_element_type=f32)
                    + b3_ref[...], 0.0)
    mlp = jnp.dot(h, w4_ref[...], preferred_element_type=f32) + b4_ref[...]
    out_ref[...] = jax.nn.sigmoid(fm + mlp)


@functools.cache
def _make_tc(batch):
    nb = batch // BM
    din = NONE_HOT * EMB
    blk = lambda i: (i, 0)
    fix = lambda i: (0, 0)
    return pl.pallas_call(
        _tc_body,
        grid=(nb,),
        in_specs=[
            pl.BlockSpec((BM, NMULTI), blk),
            pl.BlockSpec((BM, din), blk),
            pl.BlockSpec((BM, NONE_HOT), blk),
            pl.BlockSpec((NMULTI, EMB), fix),
            pl.BlockSpec((NMULTI, 1), fix),
            pl.BlockSpec((1, EMB), fix),
            pl.BlockSpec((1, 1), fix),
            pl.BlockSpec((din, EMB), fix),
            pl.BlockSpec((din, 512), fix),
            pl.BlockSpec((EMB, 512), fix),
            pl.BlockSpec((1, 512), fix),
            pl.BlockSpec((512, 256), fix),
            pl.BlockSpec((1, 256), fix),
            pl.BlockSpec((256, 128), fix),
            pl.BlockSpec((1, 128), fix),
            pl.BlockSpec((128, 1), fix),
            pl.BlockSpec((1, 1), fix),
        ],
        out_specs=pl.BlockSpec((BM, 1), blk),
        out_shape=jax.ShapeDtypeStruct((batch, 1), jnp.float32),
    )


def kernel(x, W_emb, W_fc, bias, w1, b1, w2, b2, w3, b3, w4, b4):
    batch = x.shape[0]
    din = NONE_HOT * EMB
    one_hot = x[:, :NONE_HOT]
    xm = x[:, NONE_HOT:]

    f32 = jnp.float32
    span = PACK * PACK_R
    nblk = (OFF + span - 1) // span
    ctab = _make_pack(nblk)(W_emb.T)

    fflat = W_fc.reshape(-1)

    n_idx = batch * NONE_HOT
    flat = one_hot.reshape(-1)
    gidx = ((flat // span) * PACK_R + flat % PACK_R
            ).reshape(n_idx // CHUNK, CHUNK)
    rfull = flat.reshape(n_idx // CHUNK, CHUNK)
    emb_flat = _make_sc_gather(n_idx)(gidx, rfull, ctab)
    emb = emb_flat.reshape(batch, din)
    fc2d = _make_sc_fc(n_idx)(rfull, fflat[:OFF].reshape(OFF, 1))
    fc = fc2d.reshape(batch, NONE_HOT)

    def ctab_at(r):
        return (r // span) * PACK_R + r % PACK_R, EMB * ((r // PACK_R) % PACK)
    p0, c0 = ctab_at(OFF + 1)
    wm = lax.slice(ctab, (p0, c0), (p0 + NMULTI, c0 + EMB))
    p1, c1 = ctab_at(OFF)
    pe_row = lax.slice(ctab, (p1, c1), (p1 + 1, c1 + EMB))
    wf = lax.slice(W_fc, (OFF + 1, 0), (OFF + 1 + NMULTI, 1))
    pf_row = lax.slice(W_fc, (OFF, 0), (OFF + 1, 1))
    a = jnp.tile(jnp.eye(EMB, dtype=f32), (NONE_HOT, 1))

    y = _make_tc(batch)(
        xm, emb, fc, wm, wf, pe_row, pf_row, a,
        w1[:din], w1[din:], b1.reshape(1, -1),
        w2, b2.reshape(1, -1), w3, b3.reshape(1, -1),
        w4, (b4 + bias).reshape(1, -1),
    )
    return y[:, 0]


# speedup vs baseline: 7.6731x; 7.6731x over previous
import functools

import jax
import jax.numpy as jnp
from jax import lax
from jax.experimental import pallas as pl
from jax.experimental.pallas import tpu as pltpu
from jax.experimental.pallas import tpu_sc as plsc

OFF = 2600000
NONE_HOT = 26
NMULTI = 200
EMB = 16
PACK = 128 // EMB
CHUNK = 128
PACK_R = 4096
BM = 512


def _pack_body(wt_ref, out_ref):
    cat = jnp.concatenate(
        [wt_ref[:, pl.ds(PACK_R * u, PACK_R)] for u in range(PACK)], axis=0)
    out_ref[...] = cat.T


@functools.cache
def _make_pack(nblk):
    return pl.pallas_call(
        _pack_body,
        grid=(nblk,),
        in_specs=[
            pl.BlockSpec((EMB, PACK * PACK_R), lambda i: (0, i)),
        ],
        out_specs=pl.BlockSpec((PACK_R, 128), lambda i: (i, 0)),
        out_shape=jax.ShapeDtypeStruct((nblk * PACK_R, 128), jnp.float32),
    )


def _sc_gather_body(gidx_hbm, g2_hbm, r_hbm, c_hbm, ftab, emb_hbm, fc_hbm,
                    gid_v, g2_v, r_v, be0, be1, bf0, bf1, ext, fcext,
                    se0, se1, sf0, sf1):
    nch = gidx_hbm.shape[0] // 32
    wid = lax.axis_index("s") * 2 + lax.axis_index("c")
    pltpu.sync_copy(gidx_hbm.at[pl.ds(wid * nch, nch)], gid_v)
    pltpu.sync_copy(g2_hbm.at[pl.ds(wid * nch, nch)], g2_v)
    pltpu.sync_copy(r_hbm.at[pl.ds(wid * nch, nch)], r_v)
    base = wid * nch * CHUNK
    i16 = lax.iota(jnp.int32, 16)

    def start(c, be, bf, se, sf):
        pltpu.make_async_copy(c_hbm.at[gid_v.at[c]], be, se).start()
        pltpu.make_async_copy(ftab.at[g2_v.at[c]], bf, sf).start()

    def wait(c, be, bf, se, sf):
        pltpu.make_async_copy(c_hbm.at[gid_v.at[c]], be, se).wait()
        pltpu.make_async_copy(ftab.at[g2_v.at[c]], bf, sf).wait()

    def process(c, be, bf):
        for g in range(CHUNK // 16):
            r16 = r_v[c, pl.ds(g * 16, 16)]
            rows = g * 16 + i16
            col0 = ((r16 >> (PACK_R.bit_length() - 1)) & (PACK - 1)) * EMB
            for cc in range(EMB):
                v = plsc.load_gather(be, [rows, col0 + cc])
                plsc.store_scatter(ext, [i16 * EMB + (g * 16 * EMB + cc)], v)
            fcv = plsc.load_gather(bf, [rows, r16 & 127])
            plsc.store_scatter(fcext, [c * CHUNK + g * 16 + i16], fcv)
        pltpu.sync_copy(
            ext, emb_hbm.at[pl.ds((base + c * CHUNK) * EMB, CHUNK * EMB)])

    start(0, be0, bf0, se0, sf0)

    def body(k, carry):
        c0 = 2 * k
        start(c0 + 1, be1, bf1, se1, sf1)
        wait(c0, be0, bf0, se0, sf0)
        process(c0, be0, bf0)

        @pl.when(c0 + 2 < nch)
        def _():
            start(c0 + 2, be0, bf0, se0, sf0)

        wait(c0 + 1, be1, bf1, se1, sf1)
        process(c0 + 1, be1, bf1)
        return carry

    lax.fori_loop(0, nch // 2, body, 0)
    pltpu.sync_copy(fcext, fc_hbm.at[pl.ds(base, nch * CHUNK)])


@functools.cache
def _make_sc_gather(n_idx):
    nch = n_idx // CHUNK // 32
    mesh = plsc.VectorSubcoreMesh(core_axis_name="c", subcore_axis_name="s")
    return pl.kernel(
        _sc_gather_body,
        mesh=mesh,
        compiler_params=pltpu.CompilerParams(needs_layout_passes=False),
        out_type=[
            jax.ShapeDtypeStruct((n_idx * EMB,), jnp.float32),
            jax.ShapeDtypeStruct((n_idx,), jnp.float32),
        ],
        scratch_types=[
            pltpu.VMEM((nch, CHUNK), jnp.int32),
            pltpu.VMEM((nch, CHUNK), jnp.int32),
            pltpu.VMEM((nch, CHUNK), jnp.int32),
            pltpu.VMEM((CHUNK, 128), jnp.float32),
            pltpu.VMEM((CHUNK, 128), jnp.float32),
            pltpu.VMEM((CHUNK, 128), jnp.float32),
            pltpu.VMEM((CHUNK, 128), jnp.float32),
            pltpu.VMEM((CHUNK * EMB,), jnp.float32),
            pltpu.VMEM((nch * CHUNK,), jnp.float32),
            pltpu.SemaphoreType.DMA,
            pltpu.SemaphoreType.DMA,
            pltpu.SemaphoreType.DMA,
            pltpu.SemaphoreType.DMA,
        ],
    )


def _tc_body(xm_ref, emb_ref, fc_ref, wm_ref, wf_ref, pe_ref, pf_ref, a_ref,
             w1a_ref, w1b_ref, b1_ref, w2_ref, b2_ref, w3_ref, b3_ref,
             w4_ref, b4_ref, out_ref):
    f32 = jnp.float32
    m = (xm_ref[...] != 0).astype(f32)
    me = jnp.dot(m, wm_ref[...], preferred_element_type=f32)
    mf = jnp.dot(m, wf_ref[...], preferred_element_type=f32)
    npad = float(NMULTI) - jnp.sum(m, axis=1, keepdims=True)
    me = me + npad * pe_ref[...]
    mf = mf + npad * pf_ref[...]

    emb = emb_ref[...]
    a = a_ref[...]
    s = jnp.dot(emb, a, preferred_element_type=f32) + me
    sq = jnp.dot(emb * emb, a, preferred_element_type=f32) + me * me
    fm = (jnp.sum(fc_ref[...], axis=1, keepdims=True) + mf
          + 0.5 * jnp.sum(s * s - sq, axis=1, keepdims=True))

    h = jnp.maximum(jnp.dot(emb, w1a_ref[...], preferred_element_type=f32)
                    + jnp.dot(me, w1b_ref[...], preferred_element_type=f32)
                    + b1_ref[...], 0.0)
    h = jnp.maximum(jnp.dot(h, w2_ref[...], preferred_element_type=f32)
                    + b2_ref[...], 0.0)
    h = jnp.maximum(jnp.dot(h, w3_ref[...], preferred_element_type=f32)
                    + b3_ref[...], 0.0)
    mlp = jnp.dot(h, w4_ref[...], preferred_element_type=f32) + b4_ref[...]
    out_ref[...] = jax.nn.sigmoid(fm + mlp)


@functools.cache
def _make_tc(batch):
    nb = batch // BM
    din = NONE_HOT * EMB
    blk = lambda i: (i, 0)
    fix = lambda i: (0, 0)
    return pl.pallas_call(
        _tc_body,
        grid=(nb,),
        in_specs=[
            pl.BlockSpec((BM, NMULTI), blk),
            pl.BlockSpec((BM, din), blk),
            pl.BlockSpec((BM, NONE_HOT), blk),
            pl.BlockSpec((NMULTI, EMB), fix),
            pl.BlockSpec((NMULTI, 1), fix),
            pl.BlockSpec((1, EMB), fix),
            pl.BlockSpec((1, 1), fix),
            pl.BlockSpec((din, EMB), fix),
            pl.BlockSpec((din, 512), fix),
            pl.BlockSpec((EMB, 512), fix),
            pl.BlockSpec((1, 512), fix),
            pl.BlockSpec((512, 256), fix),
            pl.BlockSpec((1, 256), fix),
            pl.BlockSpec((256, 128), fix),
            pl.BlockSpec((1, 128), fix),
            pl.BlockSpec((128, 1), fix),
            pl.BlockSpec((1, 1), fix),
        ],
        out_specs=pl.BlockSpec((BM, 1), blk),
        out_shape=jax.ShapeDtypeStruct((batch, 1), jnp.float32),
    )


def kernel(x, W_emb, W_fc, bias, w1, b1, w2, b2, w3, b3, w4, b4):
    batch = x.shape[0]
    din = NONE_HOT * EMB
    one_hot = x[:, :NONE_HOT]
    xm = x[:, NONE_HOT:]

    f32 = jnp.float32
    span = PACK * PACK_R
    nblk = (OFF + span - 1) // span
    ctab = _make_pack(nblk)(W_emb.T)

    nf = OFF // 128 + 1
    fflat = W_fc.reshape(-1)
    ftab = fflat[:nf * 128].reshape(nf, 128)

    n_idx = batch * NONE_HOT
    flat = one_hot.reshape(-1)
    gidx = ((flat // span) * PACK_R + flat % PACK_R
            ).reshape(n_idx // CHUNK, CHUNK)
    g2 = (flat // 128).reshape(n_idx // CHUNK, CHUNK)
    rfull = flat.reshape(n_idx // CHUNK, CHUNK)
    emb_flat, fc_flat = _make_sc_gather(n_idx)(gidx, g2, rfull, ctab, ftab)
    emb = emb_flat.reshape(batch, din)
    fc = fc_flat.reshape(batch, NONE_HOT)

    def ctab_at(r):
        return (r // span) * PACK_R + r % PACK_R, EMB * ((r // PACK_R) % PACK)
    p0, c0 = ctab_at(OFF + 1)
    wm = lax.slice(ctab, (p0, c0), (p0 + NMULTI, c0 + EMB))
    p1, c1 = ctab_at(OFF)
    pe_row = lax.slice(ctab, (p1, c1), (p1 + 1, c1 + EMB))
    wf = lax.slice(W_fc, (OFF + 1, 0), (OFF + 1 + NMULTI, 1))
    pf_row = lax.slice(W_fc, (OFF, 0), (OFF + 1, 1))
    a = jnp.tile(jnp.eye(EMB, dtype=f32), (NONE_HOT, 1))

    y = _make_tc(batch)(
        xm, emb, fc, wm, wf, pe_row, pf_row, a,
        w1[:din], w1[din:], b1.reshape(1, -1),
        w2, b2.reshape(1, -1), w3, b3.reshape(1, -1),
        w4, (b4 + bias).reshape(1, -1),
    )
    return y[:, 0]
